# baseline (speedup 1.0000x reference)
import jax
import jax.numpy as jnp
from jax import lax
from jax.experimental import pallas as pl
from jax.experimental.pallas import tpu as pltpu


def _siamese_kernel(s1_ref, s2_ref, table_hbm, bias_ref, out_ref,
                    tab_ref, vec_ref, sem):
    core = pl.program_id(0)
    pairs = out_ref.shape[0]
    seq = s1_ref.shape[1]

    cp = pltpu.make_async_copy(table_hbm, tab_ref.at[:, 0, :], sem)
    cp.start()
    cp.wait()

    def encode(tok_ref, slot0):
        def body(g, carry):
            for vloc in range(8):
                vec = g * 8 + vloc
                row = core * pairs + vec
                m = tab_ref[tok_ref[row, 0], 0]
                for t in range(1, seq):
                    m = jnp.maximum(m, tab_ref[tok_ref[row, t], 0])
                vec_ref[slot0 + vec, 0] = m
            return carry
        lax.fori_loop(0, pairs // 8, body, 0)

    encode(s1_ref, 0)
    encode(s2_ref, pairs)

    pooled = vec_ref[...] + bias_ref[0]
    v1 = pooled[:pairs]
    v2 = pooled[pairs:]
    eps2 = 1e-12 * 1e-12
    n1 = jnp.maximum(jnp.sum(v1 * v1, axis=2), eps2)
    n2 = jnp.maximum(jnp.sum(v2 * v2, axis=2), eps2)
    dt = jnp.sum(v1 * v2, axis=2)
    out_ref[...] = 1.0 - dt * lax.rsqrt(n1 * n2)


def kernel(table_fused, bias, sentence1, sentence2):
    v, h = table_fused.shape
    b, l = sentence1.shape
    n_cores = 2 if b % 16 == 0 else 1
    pairs = b // n_cores
    out = pl.pallas_call(
        _siamese_kernel,
        grid=(n_cores,),
        in_specs=[
            pl.BlockSpec(memory_space=pltpu.SMEM),
            pl.BlockSpec(memory_space=pltpu.SMEM),
            pl.BlockSpec(memory_space=pl.ANY),
            pl.BlockSpec((1, h), lambda i: (0, 0)),
        ],
        out_specs=pl.BlockSpec((pairs, 1), lambda i: (i, 0)),
        out_shape=jax.ShapeDtypeStruct((b, 1), jnp.float32),
        scratch_shapes=[pltpu.VMEM((v, 1, h), jnp.float32),
                        pltpu.VMEM((2 * pairs, 1, h), jnp.float32),
                        pltpu.SemaphoreType.DMA],
        compiler_params=pltpu.CompilerParams(
            dimension_semantics=("parallel",),
            vmem_limit_bytes=32 * 1024 * 1024),
    )(sentence1.astype(jnp.int32), sentence2.astype(jnp.int32),
      table_fused, bias)
    return out.reshape(-1)


# speedup vs baseline: 1.1943x; 1.1943x over previous
import jax
import jax.numpy as jnp
from jax import lax
from jax.experimental import pallas as pl
from jax.experimental.pallas import tpu as pltpu


def _siamese_kernel(s1_ref, s2_ref, table_hbm, bias_ref, out_ref,
                    tab_ref, vec_ref, sem):
    core = pl.program_id(0)
    pairs = out_ref.shape[0]
    seq = s1_ref.shape[1]

    n_dma = 8
    vchunk = tab_ref.shape[0] // n_dma
    for k in range(n_dma):
        pltpu.make_async_copy(
            table_hbm.at[pl.ds(k * vchunk, vchunk), :],
            tab_ref.at[pl.ds(k * vchunk, vchunk), 0, :],
            sem.at[k]).start()
    for k in range(n_dma):
        pltpu.make_async_copy(
            table_hbm.at[pl.ds(k * vchunk, vchunk), :],
            tab_ref.at[pl.ds(k * vchunk, vchunk), 0, :],
            sem.at[k]).wait()

    def encode(tok_ref, slot0):
        def body(g, carry):
            for vloc in range(8):
                vec = g * 8 + vloc
                row = core * pairs + vec
                m = tab_ref[tok_ref[row, 0], 0]
                for t in range(1, seq):
                    m = jnp.maximum(m, tab_ref[tok_ref[row, t], 0])
                vec_ref[slot0 + vec, 0] = m
            return carry
        lax.fori_loop(0, pairs // 8, body, 0)

    encode(s1_ref, 0)
    encode(s2_ref, pairs)

    pooled = vec_ref[...] + bias_ref[0]
    v1 = pooled[:pairs]
    v2 = pooled[pairs:]
    eps2 = 1e-12 * 1e-12
    n1 = jnp.maximum(jnp.sum(v1 * v1, axis=2), eps2)
    n2 = jnp.maximum(jnp.sum(v2 * v2, axis=2), eps2)
    dt = jnp.sum(v1 * v2, axis=2)
    out_ref[...] = 1.0 - dt * lax.rsqrt(n1 * n2)


def kernel(table_fused, bias, sentence1, sentence2):
    v, h = table_fused.shape
    b, l = sentence1.shape
    n_cores = 2 if b % 16 == 0 else 1
    pairs = b // n_cores
    out = pl.pallas_call(
        _siamese_kernel,
        grid=(n_cores,),
        in_specs=[
            pl.BlockSpec(memory_space=pltpu.SMEM),
            pl.BlockSpec(memory_space=pltpu.SMEM),
            pl.BlockSpec(memory_space=pl.ANY),
            pl.BlockSpec((1, h), lambda i: (0, 0)),
        ],
        out_specs=pl.BlockSpec((pairs, 1), lambda i: (i, 0)),
        out_shape=jax.ShapeDtypeStruct((b, 1), jnp.float32),
        scratch_shapes=[pltpu.VMEM((v, 1, h), jnp.float32),
                        pltpu.VMEM((2 * pairs, 1, h), jnp.float32),
                        pltpu.SemaphoreType.DMA((8,))],
        compiler_params=pltpu.CompilerParams(
            dimension_semantics=("parallel",),
            vmem_limit_bytes=32 * 1024 * 1024),
    )(sentence1.astype(jnp.int32), sentence2.astype(jnp.int32),
      table_fused, bias)
    return out.reshape(-1)


# speedup vs baseline: 1.2801x; 1.0718x over previous
import jax
import jax.numpy as jnp
from jax import lax
from jax.experimental import pallas as pl
from jax.experimental.pallas import tpu as pltpu


def _siamese_kernel(s1_ref, s2_ref, table_hbm, bias_ref, out_ref,
                    tab_ref, vec_ref, sem):
    core = pl.program_id(0)
    pairs = out_ref.shape[0]
    seq = s1_ref.shape[1]

    n_dma = 8
    vchunk = tab_ref.shape[0] // n_dma
    for k in range(n_dma):
        pltpu.make_async_copy(
            table_hbm.at[pl.ds(k * vchunk, vchunk)],
            tab_ref.at[pl.ds(k * vchunk, vchunk)],
            sem.at[k]).start()
    for k in range(n_dma):
        pltpu.make_async_copy(
            table_hbm.at[pl.ds(k * vchunk, vchunk)],
            tab_ref.at[pl.ds(k * vchunk, vchunk)],
            sem.at[k]).wait()

    def encode(tok_ref, slot0):
        def body(g, carry):
            for vloc in range(8):
                vec = g * 8 + vloc
                row = core * pairs + vec
                m = tab_ref[tok_ref[row, 0]]
                for t in range(1, seq):
                    m = jnp.maximum(m, tab_ref[tok_ref[row, t]])
                vec_ref[slot0 + vec] = m
            return carry
        lax.fori_loop(0, pairs // 8, body, 0)

    encode(s1_ref, 0)
    encode(s2_ref, pairs)

    pooled = vec_ref[...] + bias_ref[...]
    v1 = pooled[:pairs]
    v2 = pooled[pairs:]
    eps2 = 1e-12 * 1e-12
    n1 = jnp.maximum(jnp.sum(jnp.sum(v1 * v1, axis=2), axis=1,
                             keepdims=True), eps2)
    n2 = jnp.maximum(jnp.sum(jnp.sum(v2 * v2, axis=2), axis=1,
                             keepdims=True), eps2)
    dt = jnp.sum(jnp.sum(v1 * v2, axis=2), axis=1, keepdims=True)
    out_ref[...] = 1.0 - dt * lax.rsqrt(n1 * n2)


def kernel(table_fused, bias, sentence1, sentence2):
    v, h = table_fused.shape
    b, l = sentence1.shape
    lanes = 128
    s = h // lanes
    n_cores = 2 if b % 16 == 0 else 1
    pairs = b // n_cores
    out = pl.pallas_call(
        _siamese_kernel,
        grid=(n_cores,),
        in_specs=[
            pl.BlockSpec(memory_space=pltpu.SMEM),
            pl.BlockSpec(memory_space=pltpu.SMEM),
            pl.BlockSpec(memory_space=pl.ANY),
            pl.BlockSpec((s, lanes), lambda i: (0, 0)),
        ],
        out_specs=pl.BlockSpec((pairs, 1), lambda i: (i, 0)),
        out_shape=jax.ShapeDtypeStruct((b, 1), jnp.float32),
        scratch_shapes=[pltpu.VMEM((v, s, lanes), jnp.float32),
                        pltpu.VMEM((2 * pairs, s, lanes), jnp.float32),
                        pltpu.SemaphoreType.DMA((8,))],
        compiler_params=pltpu.CompilerParams(
            dimension_semantics=("parallel",),
            vmem_limit_bytes=32 * 1024 * 1024),
    )(sentence1.astype(jnp.int32), sentence2.astype(jnp.int32),
      table_fused.reshape(v, s, lanes), bias.reshape(s, lanes))
    return out.reshape(-1)


# speedup vs baseline: 1.3869x; 1.0835x over previous
import jax
import jax.numpy as jnp
from jax import lax
from jax.experimental import pallas as pl
from jax.experimental.pallas import tpu as pltpu


def _siamese_kernel(s1_ref, s2_ref, tab_ref, bias_ref, out_ref, vec_ref):
    core = pl.program_id(0)
    pairs = out_ref.shape[0]
    seq = s1_ref.shape[1]

    def chunk_rot(tok):
        c = pl.multiple_of((tok >> 3) << 3, 8)
        ch = tab_ref[pl.ds(c, 8), :]
        return pltpu.roll(ch, 0 - (tok & 7), axis=0)

    def encode(tok_ref, slot0):
        def body(g, carry):
            rows = []
            for vloc in range(8):
                row = core * pairs + g * 8 + vloc
                acc = chunk_rot(tok_ref[row, 0])
                for t in range(1, seq):
                    acc = jnp.maximum(acc, chunk_rot(tok_ref[row, t]))
                rows.append(acc[0:1, :])
            blk = jnp.concatenate(rows, axis=0)
            vec_ref[pl.ds(pl.multiple_of(slot0 + g * 8, 8), 8), :] = blk
            return carry
        lax.fori_loop(0, pairs // 8, body, 0)

    encode(s1_ref, 0)
    encode(s2_ref, pairs)

    pooled = vec_ref[...] + bias_ref[...]
    v1 = pooled[:pairs]
    v2 = pooled[pairs:]
    eps2 = 1e-12 * 1e-12
    n1 = jnp.maximum(jnp.sum(v1 * v1, axis=1, keepdims=True), eps2)
    n2 = jnp.maximum(jnp.sum(v2 * v2, axis=1, keepdims=True), eps2)
    dt = jnp.sum(v1 * v2, axis=1, keepdims=True)
    out_ref[...] = 1.0 - dt * lax.rsqrt(n1 * n2)


def kernel(table_fused, bias, sentence1, sentence2):
    v, h = table_fused.shape
    b, l = sentence1.shape
    n_cores = 2 if b % 16 == 0 else 1
    pairs = b // n_cores
    out = pl.pallas_call(
        _siamese_kernel,
        grid=(n_cores,),
        in_specs=[
            pl.BlockSpec(memory_space=pltpu.SMEM),
            pl.BlockSpec(memory_space=pltpu.SMEM),
            pl.BlockSpec((v, h), lambda i: (0, 0)),
            pl.BlockSpec((1, h), lambda i: (0, 0)),
        ],
        out_specs=pl.BlockSpec((pairs, 1), lambda i: (i, 0)),
        out_shape=jax.ShapeDtypeStruct((b, 1), jnp.float32),
        scratch_shapes=[pltpu.VMEM((2 * pairs, h), jnp.float32)],
        compiler_params=pltpu.CompilerParams(
            dimension_semantics=("parallel",),
            vmem_limit_bytes=48 * 1024 * 1024),
    )(sentence1.astype(jnp.int32), sentence2.astype(jnp.int32),
      table_fused, bias)
    return out.reshape(-1)


# speedup vs baseline: 1.4979x; 1.0801x over previous
import jax
import jax.numpy as jnp
from jax import lax
from jax.experimental import pallas as pl
from jax.experimental.pallas import tpu as pltpu


def _siamese_kernel(c1_ref, c2_ref, table_hbm, bias_ref,
                    out_ref, tab_ref, vec_ref, sem):
    core = pl.program_id(0)
    pairs = out_ref.shape[0]
    seq = c1_ref.shape[1]
    vc, sub, h = tab_ref.shape

    table3 = table_hbm.reshape(vc, sub, h)
    n_dma = 8
    chunk = vc // n_dma
    for k in range(n_dma):
        pltpu.make_async_copy(
            table3.at[pl.ds(k * chunk, chunk)],
            tab_ref.at[pl.ds(k * chunk, chunk)],
            sem.at[k]).start()
    for k in range(n_dma):
        pltpu.make_async_copy(
            table3.at[pl.ds(k * chunk, chunk)],
            tab_ref.at[pl.ds(k * chunk, chunk)],
            sem.at[k]).wait()

    def encode(c_ref, slot0):
        def body(g, carry):
            rows = []
            for vloc in range(8):
                row = core * pairs + g * 8 + vloc

                def fetch(t):
                    cr = c_ref[row, t]
                    return pltpu.roll(tab_ref[cr >> 3], cr & 7, axis=0)

                acc = fetch(0)
                for t in range(1, seq):
                    acc = jnp.maximum(acc, fetch(t))
                rows.append(acc[0:1, :])
            blk = jnp.concatenate(rows, axis=0)
            vec_ref[pl.ds(pl.multiple_of(slot0 + g * 8, 8), 8), :] = blk
            return carry
        lax.fori_loop(0, pairs // 8, body, 0)

    encode(c1_ref, 0)
    encode(c2_ref, pairs)

    pooled = vec_ref[...] + bias_ref[...]
    v1 = pooled[:pairs]
    v2 = pooled[pairs:]
    eps2 = 1e-12 * 1e-12
    n1 = jnp.maximum(jnp.sum(v1 * v1, axis=1, keepdims=True), eps2)
    n2 = jnp.maximum(jnp.sum(v2 * v2, axis=1, keepdims=True), eps2)
    dt = jnp.sum(v1 * v2, axis=1, keepdims=True)
    out_ref[...] = 1.0 - dt * lax.rsqrt(n1 * n2)


def kernel(table_fused, bias, sentence1, sentence2):
    v, h = table_fused.shape
    b, l = sentence1.shape
    n_cores = 2 if b % 16 == 0 else 1
    pairs = b // n_cores
    s1 = sentence1.astype(jnp.int32)
    s2 = sentence2.astype(jnp.int32)
    call = pl.pallas_call(
        _siamese_kernel,
        grid=(n_cores,),
        in_specs=[
            pl.BlockSpec(memory_space=pltpu.SMEM),
            pl.BlockSpec(memory_space=pltpu.SMEM),
            pl.BlockSpec(memory_space=pltpu.MemorySpace.HBM),
            pl.BlockSpec((1, h), lambda i: (0, 0)),
        ],
        out_specs=pl.BlockSpec((pairs, 1), lambda i: (i, 0)),
        out_shape=jax.ShapeDtypeStruct((b, 1), jnp.float32),
        scratch_shapes=[pltpu.VMEM((v // 8, 8, h), jnp.float32),
                        pltpu.VMEM((2 * pairs, h), jnp.float32),
                        pltpu.SemaphoreType.DMA((8,))],
        compiler_params=pltpu.CompilerParams(
            dimension_semantics=("parallel",),
            vmem_limit_bytes=32 * 1024 * 1024),
    )
    c1 = ((s1 >> 3) << 3) | ((8 - (s1 & 7)) & 7)
    c2 = ((s2 >> 3) << 3) | ((8 - (s2 & 7)) & 7)
    out = call(c1, c2, table_fused, bias)
    return out.reshape(-1)


# speedup vs baseline: 1.8340x; 1.2243x over previous
import jax
import jax.numpy as jnp
from jax import lax
from jax.experimental import pallas as pl
from jax.experimental.pallas import tpu as pltpu


def _siamese_kernel(c1_ref, c2_ref, table_ref, bias_ref,
                    out_ref, vec_ref):
    core = pl.program_id(0)
    pairs = out_ref.shape[0]
    seq = c1_ref.shape[1]
    v, h = table_ref.shape

    tab = table_ref.reshape(v // 8, 8, h)

    def encode(c_ref, slot0):
        def body(g, carry):
            rows = []
            for vloc in range(8):
                row = core * pairs + g * 8 + vloc

                def fetch(t):
                    cr = c_ref[row, t]
                    return pltpu.roll(tab[cr >> 3], cr & 7, axis=0)

                acc = fetch(0)
                for t in range(1, seq):
                    acc = jnp.maximum(acc, fetch(t))
                rows.append(acc[0:1, :])
            blk = jnp.concatenate(rows, axis=0)
            vec_ref[pl.ds(pl.multiple_of(slot0 + g * 8, 8), 8), :] = blk
            return carry
        lax.fori_loop(0, pairs // 8, body, 0)

    encode(c1_ref, 0)
    encode(c2_ref, pairs)

    pooled = vec_ref[...] + bias_ref[...]
    v1 = pooled[:pairs]
    v2 = pooled[pairs:]
    eps2 = 1e-12 * 1e-12
    n1 = jnp.maximum(jnp.sum(v1 * v1, axis=1, keepdims=True), eps2)
    n2 = jnp.maximum(jnp.sum(v2 * v2, axis=1, keepdims=True), eps2)
    dt = jnp.sum(v1 * v2, axis=1, keepdims=True)
    out_ref[...] = 1.0 - dt * lax.rsqrt(n1 * n2)


def kernel(table_fused, bias, sentence1, sentence2):
    v, h = table_fused.shape
    b, l = sentence1.shape
    n_cores = 2 if b % 16 == 0 else 1
    pairs = b // n_cores
    s1 = sentence1.astype(jnp.int32)
    s2 = sentence2.astype(jnp.int32)
    call = pl.pallas_call(
        _siamese_kernel,
        grid=(n_cores,),
        in_specs=[
            pl.BlockSpec(memory_space=pltpu.SMEM),
            pl.BlockSpec(memory_space=pltpu.SMEM),
            pl.BlockSpec(memory_space=pltpu.MemorySpace.VMEM),
            pl.BlockSpec((1, h), lambda i: (0, 0)),
        ],
        out_specs=pl.BlockSpec((pairs, 1), lambda i: (i, 0)),
        out_shape=jax.ShapeDtypeStruct((b, 1), jnp.float32),
        scratch_shapes=[pltpu.VMEM((2 * pairs, h), jnp.float32)],
        compiler_params=pltpu.CompilerParams(
            dimension_semantics=("parallel",),
            vmem_limit_bytes=48 * 1024 * 1024),
    )
    c1 = ((s1 >> 3) << 3) | ((8 - (s1 & 7)) & 7)
    c2 = ((s2 >> 3) << 3) | ((8 - (s2 & 7)) & 7)
    out = call(c1, c2, table_fused, bias)
    return out.reshape(-1)


# speedup vs baseline: 1.9252x; 1.0498x over previous
import jax
import jax.numpy as jnp
from jax import lax
from jax.experimental import pallas as pl
from jax.experimental.pallas import tpu as pltpu


def _siamese_kernel(c1_ref, c2_ref, table_ref, bias_ref,
                    out_ref, vec_ref):
    core = pl.program_id(0)
    ogroups = out_ref.shape[0]
    pairs = ogroups * 4
    seq = c1_ref.shape[1]
    v, h = table_ref.shape
    vpg = 4

    tab = table_ref.reshape(v // 8, 8, h)

    def encode(c_ref, gslot0):
        def body(g, carry):
            rows = []
            for vloc in range(vpg):
                row = core * pairs + g * vpg + vloc

                def fetch(t):
                    cr = c_ref[row, t]
                    return pltpu.roll(tab[cr >> 3], cr & 7, axis=0)

                acc = fetch(0)
                for t in range(1, seq):
                    acc = jnp.maximum(acc, fetch(t))
                rows.append(acc[0:1, :])
            vec_ref[gslot0 + g] = jnp.concatenate(rows, axis=0)
            return carry
        lax.fori_loop(0, pairs // vpg, body, 0)

    encode(c1_ref, 0)
    encode(c2_ref, pairs // vpg)

    pooled = vec_ref[...] + bias_ref[...]
    v1 = pooled[:ogroups]
    v2 = pooled[ogroups:]
    eps2 = 1e-12 * 1e-12
    n1 = jnp.maximum(jnp.sum(v1 * v1, axis=2), eps2)
    n2 = jnp.maximum(jnp.sum(v2 * v2, axis=2), eps2)
    dt = jnp.sum(v1 * v2, axis=2)
    out_ref[...] = 1.0 - dt * lax.rsqrt(n1 * n2)


def kernel(table_fused, bias, sentence1, sentence2):
    v, h = table_fused.shape
    b, l = sentence1.shape
    n_cores = 2 if b % 16 == 0 else 1
    pairs = b // n_cores
    s1 = sentence1.astype(jnp.int32)
    s2 = sentence2.astype(jnp.int32)
    call = pl.pallas_call(
        _siamese_kernel,
        grid=(n_cores,),
        in_specs=[
            pl.BlockSpec(memory_space=pltpu.SMEM),
            pl.BlockSpec(memory_space=pltpu.SMEM),
            pl.BlockSpec(memory_space=pltpu.MemorySpace.VMEM),
            pl.BlockSpec((1, h), lambda i: (0, 0)),
        ],
        out_specs=pl.BlockSpec((pairs // 4, 4), lambda i: (i, 0)),
        out_shape=jax.ShapeDtypeStruct((b // 4, 4), jnp.float32),
        scratch_shapes=[pltpu.VMEM((2 * pairs // 4, 4, h), jnp.float32)],
        compiler_params=pltpu.CompilerParams(
            dimension_semantics=("parallel",),
            vmem_limit_bytes=48 * 1024 * 1024),
    )
    c1 = ((s1 >> 3) << 3) | ((8 - (s1 & 7)) & 7)
    c2 = ((s2 >> 3) << 3) | ((8 - (s2 & 7)) & 7)
    out = call(c1, c2, table_fused, bias)
    return out.reshape(-1)
